# baseline (speedup 1.0000x reference)
import functools

import jax
import jax.numpy as jnp
from jax.experimental import pallas as pl
from jax.experimental.pallas import tpu as pltpu

SMOOTH = 0.1
VOCAB = 32000
FILL = SMOOTH / (VOCAB - 1)
CONF = 1.0 - SMOOTH
import math
C_CONST = SMOOTH * math.log(FILL) + CONF * math.log(CONF)

ROWS_PER_BLOCK = 64


def _loss_kernel(pred_ref, tgt_ref, out_ref, acc_ref, cnt_ref, *, num_blocks):
    i = pl.program_id(0)

    pred = pred_ref[...]
    tgt = tgt_ref[...]
    col = jax.lax.broadcasted_iota(jnp.int32, pred.shape, 1)
    w = jnp.where(col == tgt, CONF, FILL)
    wsum = jnp.sum(w * pred, axis=1, keepdims=True)
    mask = (tgt != 0).astype(jnp.float32)
    block_loss = jnp.sum(mask * (C_CONST - wsum))
    block_cnt = jnp.sum(mask)

    @pl.when(i == 0)
    def _():
        acc_ref[0, 0] = 0.0
        cnt_ref[0, 0] = 0.0

    acc_ref[0, 0] += block_loss
    cnt_ref[0, 0] += block_cnt

    @pl.when(i == num_blocks - 1)
    def _():
        out_ref[...] = jnp.full((1, 1), acc_ref[0, 0] / cnt_ref[0, 0],
                                dtype=jnp.float32)


def kernel(prediction, target):
    _, n_tok, vocab = prediction.shape
    pred2d = prediction.reshape(n_tok, vocab)
    tgt_col = target.reshape(n_tok, 1)
    num_blocks = n_tok // ROWS_PER_BLOCK

    out = pl.pallas_call(
        functools.partial(_loss_kernel, num_blocks=num_blocks),
        grid=(num_blocks,),
        in_specs=[
            pl.BlockSpec((ROWS_PER_BLOCK, vocab), lambda i: (i, 0)),
            pl.BlockSpec((ROWS_PER_BLOCK, 1), lambda i: (i, 0)),
        ],
        out_specs=pl.BlockSpec((1, 1), lambda i: (0, 0)),
        out_shape=jax.ShapeDtypeStruct((1, 1), jnp.float32),
        scratch_shapes=[
            pltpu.SMEM((1, 1), jnp.float32),
            pltpu.SMEM((1, 1), jnp.float32),
        ],
    )(pred2d, tgt_col)
    return out[0, 0]


# speedup vs baseline: 1.0238x; 1.0238x over previous
import functools

import jax
import jax.numpy as jnp
from jax.experimental import pallas as pl
from jax.experimental.pallas import tpu as pltpu

SMOOTH = 0.1
VOCAB = 32000
FILL = SMOOTH / (VOCAB - 1)
CONF = 1.0 - SMOOTH
import math
C_CONST = SMOOTH * math.log(FILL) + CONF * math.log(CONF)

ROWS_PER_BLOCK = 128


def _loss_kernel(pred_lo_ref, pred_hi_ref, tgt_ref, out_ref, acc_ref, cnt_ref,
                 *, num_blocks, half):
    i = pl.program_id(0)

    tgt = tgt_ref[...]
    lo = pred_lo_ref[...]
    hi = pred_hi_ref[...]
    col = jax.lax.broadcasted_iota(jnp.int32, lo.shape, 1)
    w_lo = jnp.where(col == tgt, CONF, FILL)
    w_hi = jnp.where(col + half == tgt, CONF, FILL)
    wsum = (jnp.sum(w_lo * lo, axis=1, keepdims=True)
            + jnp.sum(w_hi * hi, axis=1, keepdims=True))
    mask = (tgt != 0).astype(jnp.float32)
    block_loss = jnp.sum(mask * (C_CONST - wsum))
    block_cnt = jnp.sum(mask)

    @pl.when(i == 0)
    def _():
        acc_ref[0, 0] = 0.0
        cnt_ref[0, 0] = 0.0

    acc_ref[0, 0] += block_loss
    cnt_ref[0, 0] += block_cnt

    @pl.when(i == num_blocks - 1)
    def _():
        out_ref[...] = jnp.full((1, 1), acc_ref[0, 0] / cnt_ref[0, 0],
                                dtype=jnp.float32)


def kernel(prediction, target):
    _, n_tok, vocab = prediction.shape
    pred2d = prediction.reshape(n_tok, vocab)
    tgt_col = target.reshape(n_tok, 1)
    num_blocks = n_tok // ROWS_PER_BLOCK

    half = vocab // 2
    out = pl.pallas_call(
        functools.partial(_loss_kernel, num_blocks=num_blocks, half=half),
        grid=(num_blocks,),
        in_specs=[
            pl.BlockSpec((ROWS_PER_BLOCK, half), lambda i: (i, 0)),
            pl.BlockSpec((ROWS_PER_BLOCK, half), lambda i: (i, 1)),
            pl.BlockSpec((ROWS_PER_BLOCK, 1), lambda i: (i, 0)),
        ],
        out_specs=pl.BlockSpec((1, 1), lambda i: (0, 0)),
        out_shape=jax.ShapeDtypeStruct((1, 1), jnp.float32),
        scratch_shapes=[
            pltpu.SMEM((1, 1), jnp.float32),
            pltpu.SMEM((1, 1), jnp.float32),
        ],
    )(pred2d, pred2d, tgt_col)
    return out[0, 0]
